# baseline (speedup 1.0000x reference)
import jax
import jax.numpy as jnp
from jax import lax
from jax.experimental import pallas as pl
from jax.experimental.pallas import tpu as pltpu

N_DOMAIN = 8
EPS = 1e-05
ROWS = 16384
COLS = 1024
BR = 4096
NB = ROWS // BR
COLH = 512
NH = COLS // COLH


def _onehot_t(y_ref, i):
    yv = y_ref[i]
    ids = lax.broadcasted_iota(jnp.int32, (N_DOMAIN, BR), 0)
    return (ids == yv).astype(jnp.float32)


def _fused_kernel(y_ref, g_ref, b_ref, x_any, out_ref,
                  xbuf, sums, sumsq, cnt, atab, btab, sems):
    h = pl.program_id(0)
    p = pl.program_id(1)
    i = pl.program_id(2)

    @pl.when(p == 0)
    def _phase0():
        @pl.when(jnp.logical_and(h == 0, i == 0))
        def _first():
            pltpu.make_async_copy(
                x_any.at[pl.ds(0, BR), pl.ds(0, COLH)],
                xbuf.at[pl.ds(0, BR), :], sems.at[0]).start()

        @pl.when(i == 0)
        def _zero():
            sums[...] = jnp.zeros_like(sums)
            sumsq[...] = jnp.zeros_like(sumsq)
            cnt[...] = jnp.zeros_like(cnt)

        @pl.when(i + 1 < NB)
        def _next():
            pltpu.make_async_copy(
                x_any.at[pl.ds((i + 1) * BR, BR), pl.ds(h * COLH, COLH)],
                xbuf.at[pl.ds((i + 1) * BR, BR), :], sems.at[i + 1]).start()

        pltpu.make_async_copy(
            x_any.at[pl.ds(i * BR, BR), pl.ds(h * COLH, COLH)],
            xbuf.at[pl.ds(i * BR, BR), :], sems.at[i]).wait()

        xb = xbuf[pl.ds(i * BR, BR), :]
        oh = _onehot_t(y_ref, i)
        sums[...] += lax.dot_general(
            oh, xb, (((1,), (0,)), ((), ())),
            preferred_element_type=jnp.float32)
        sumsq[...] += lax.dot_general(
            oh, xb * xb, (((1,), (0,)), ((), ())),
            preferred_element_type=jnp.float32)
        cnt[...] += jnp.broadcast_to(
            jnp.sum(oh, axis=1, keepdims=True), cnt.shape)

    @pl.when(p == 1)
    def _phase1():
        @pl.when(i == 0)
        def _tables():
            c = cnt[:, :1]
            denom = jnp.maximum(c, 1.0)
            mean = sums[...] / denom
            var = jnp.maximum(sumsq[...] / denom - mean * mean, 0.0)
            gh = g_ref[:, pl.ds(h * COLH, COLH)]
            bh = b_ref[:, pl.ds(h * COLH, COLH)]
            scale = gh * lax.rsqrt(var + EPS)
            multi = c > 1.0
            atab[...] = jnp.where(multi, scale, 1.0)
            btab[...] = jnp.where(multi, bh - mean * scale, 0.0)

        oh = _onehot_t(y_ref, i)
        row_a = lax.dot_general(oh, atab[...], (((0,), (0,)), ((), ())),
                                preferred_element_type=jnp.float32)
        row_b = lax.dot_general(oh, btab[...], (((0,), (0,)), ((), ())),
                                preferred_element_type=jnp.float32)
        out_ref[...] = xbuf[pl.ds(i * BR, BR), :] * row_a + row_b

        @pl.when(jnp.logical_and(i == NB - 1, h + 1 < NH))
        def _prefetch_next_half():
            pltpu.make_async_copy(
                x_any.at[pl.ds(0, BR), pl.ds((h + 1) * COLH, COLH)],
                xbuf.at[pl.ds(0, BR), :], sems.at[0]).start()


@jax.jit
def kernel(x, y, gamma, beta):
    y3 = y.astype(jnp.int32).reshape(NB, 1, BR)
    out = pl.pallas_call(
        _fused_kernel,
        grid=(NH, 2, NB),
        in_specs=[
            pl.BlockSpec((NB, 1, BR), lambda h, p, i: (0, 0, 0)),
            pl.BlockSpec((1, COLS), lambda h, p, i: (0, 0)),
            pl.BlockSpec((1, COLS), lambda h, p, i: (0, 0)),
            pl.BlockSpec(memory_space=pl.ANY),
        ],
        out_specs=pl.BlockSpec((BR, COLH), lambda h, p, i: (i * p, h)),
        out_shape=jax.ShapeDtypeStruct((ROWS, COLS), jnp.float32),
        scratch_shapes=[
            pltpu.VMEM((ROWS, COLH), jnp.float32),
            pltpu.VMEM((N_DOMAIN, COLH), jnp.float32),
            pltpu.VMEM((N_DOMAIN, COLH), jnp.float32),
            pltpu.VMEM((N_DOMAIN, 128), jnp.float32),
            pltpu.VMEM((N_DOMAIN, COLH), jnp.float32),
            pltpu.VMEM((N_DOMAIN, COLH), jnp.float32),
            pltpu.SemaphoreType.DMA((NB,)),
        ],
    )(y3, gamma, beta, x)
    return out
